# baseline (speedup 1.0000x reference)
import functools

import jax
import jax.numpy as jnp
from jax import lax
from jax.experimental import pallas as pl
from jax.experimental.pallas import tpu as pltpu
from jax.experimental.pallas import tpu_sc as plsc

B = 4096
L = 200
V = 40961
D = 128
C0 = 104
C1 = 96
NW = 32
P = 4
NP = B // P
ROWS = 512


def _make_bag_sum(piece):
    spw = NP // NW

    def body(xw_hbm, xb_hbm, emb_hbm, out_hbm,
             idxw_v, idxb_v, rb0, rb1, out_v, sem0, sem1):
        w = lax.axis_index("s") * 2 + lax.axis_index("c")
        base = piece * NP + w * spw

        pltpu.sync_copy(xw_hbm.at[pl.ds(base, spw)], idxw_v)
        pltpu.sync_copy(xb_hbm.at[pl.ds(base, spw)], idxb_v)

        rbufs = (rb0, rb1)
        sems = (sem0, sem1)
        counts = (C0, C1)
        offs = (0, C0)

        def wait(b):
            pltpu.make_async_copy(
                emb_hbm.at[idxw_v.at[0, pl.ds(offs[b], counts[b])]],
                rbufs[b], sems[b]).wait()

        def accum(b, init):
            rbuf = rbufs[b]

            def rbody(j, a):
                word = rbuf[j, :]
                lo = jax.lax.bitcast_convert_type(word << 16, jnp.float32)
                hi = jax.lax.bitcast_convert_type(
                    word & jnp.int32(-65536), jnp.float32)
                return (a[0] + lo, a[1] + hi)

            return lax.fori_loop(0, counts[b], rbody, init, unroll=4)

        zeros = (jnp.zeros((D // 2,), jnp.float32),
                 jnp.zeros((D // 2,), jnp.float32))
        nch = 2 * spw

        def phase(idx_v, bag):
            def issue(c, b):
                pltpu.make_async_copy(
                    emb_hbm.at[idx_v.at[c >> 1, pl.ds(offs[b], counts[b])]],
                    rbufs[b], sems[b]).start()

            issue(0, 0)
            issue(1, 1)

            def outer(i, _):
                c = 2 * i
                wait(0)
                acc = accum(0, zeros)

                @pl.when(c + 2 < nch)
                def _():
                    issue(c + 2, 0)

                wait(1)
                acc = accum(1, acc)

                @pl.when(c + 3 < nch)
                def _():
                    issue(c + 3, 1)

                out_v[i, pl.ds(bag * D, D // 2)] = acc[0]
                out_v[i, pl.ds(bag * D + D // 2, D // 2)] = acc[1]
                return 0

            lax.fori_loop(0, spw, outer, 0)

        phase(idxw_v, 0)
        phase(idxb_v, 1)

        pltpu.sync_copy(out_v, out_hbm.at[pl.ds(w * spw, spw)])

    return functools.partial(
        pl.kernel,
        out_type=jax.ShapeDtypeStruct((NP, 2 * D), jnp.float32),
        mesh=plsc.VectorSubcoreMesh(core_axis_name="c", subcore_axis_name="s"),
        scratch_types=[
            pltpu.VMEM((spw, L), jnp.int32),
            pltpu.VMEM((spw, L), jnp.int32),
            pltpu.VMEM((C0, D // 2), jnp.int32),
            pltpu.VMEM((C1, D // 2), jnp.int32),
            pltpu.VMEM((spw, 2 * D), jnp.float32),
            pltpu.SemaphoreType.DMA,
            pltpu.SemaphoreType.DMA,
        ],
        compiler_params=pltpu.CompilerParams(use_tc_tiling_on_sc=False),
    )(body)


_bag_sum_pieces = [_make_bag_sum(p) for p in range(P)]


def _heads_piece(p, pooled, w1t, b1, ew1, eb1, ew2, eb2, ew3, eb3, prev):
    grid = (NP // ROWS,)
    off = p * (NP // ROWS)
    full = lambda shape: pl.BlockSpec(shape, lambda i: (0, 0))
    in_specs = [
        pl.BlockSpec((ROWS, 2 * D), lambda i: (i, 0)),
        full((2 * D, 4096)),
        full((1, 4096)),
        full((2 * D, D)),
        full((1, D)),
        full((D, D)),
        full((1, D)),
        full((D, D)),
        full((1, D)),
    ]
    args = [pooled, w1t, b1, ew1, eb1, ew2, eb2, ew3, eb3]
    aliases = {}
    if prev is not None:
        in_specs += [pl.BlockSpec(memory_space=pl.ANY),
                     pl.BlockSpec(memory_space=pl.ANY)]
        args += [prev[0], prev[1]]
        aliases = {9: 0, 10: 1}

    def body(*refs):
        (pooled_ref, w1t_ref, b1_ref, ew1_ref, eb1_ref,
         ew2_ref, eb2_ref, ew3_ref, eb3_ref) = refs[:9]
        y_ref, z_ref = refs[len(in_specs):]
        h = jnp.maximum(pooled_ref[...], 0.0)
        hp = jax.lax.Precision.DEFAULT
        y_ref[...] = (jnp.dot(h, w1t_ref[...], precision=hp,
                              preferred_element_type=jnp.float32)
                      + b1_ref[...])
        z1 = jnp.maximum(jnp.dot(h, ew1_ref[...], precision=hp,
                                 preferred_element_type=jnp.float32)
                         + eb1_ref[...], 0.0)
        z2 = jnp.maximum(jnp.dot(z1, ew2_ref[...], precision=hp,
                                 preferred_element_type=jnp.float32)
                         + eb2_ref[...], 0.0)
        z_ref[...] = (jnp.dot(z2, ew3_ref[...], precision=hp,
                              preferred_element_type=jnp.float32)
                      + eb3_ref[...])

    return pl.pallas_call(
        body,
        grid=grid,
        in_specs=in_specs,
        out_specs=[
            pl.BlockSpec((ROWS, 4096), lambda i: (off + i, 0)),
            pl.BlockSpec((ROWS, D), lambda i: (off + i, 0)),
        ],
        out_shape=[
            jax.ShapeDtypeStruct((B, 4096), jnp.float32),
            jax.ShapeDtypeStruct((B, D), jnp.float32),
        ],
        input_output_aliases=aliases,
    )(*args)


def kernel(x_w, x_b, emb, m_w1, m_b1, e_w1, e_b1, e_w2, e_b2, e_w3, e_b3):
    r = jax.lax.bitcast_convert_type(emb, jnp.uint32) + jnp.uint32(0x8000)
    w = (r[:, :64] >> 16) | (r[:, 64:] & jnp.uint32(0xFFFF0000))
    tab = jax.lax.bitcast_convert_type(w, jnp.int32)

    w1t = m_w1.T
    b1 = m_b1.reshape(1, 4096)
    ew1 = jnp.zeros((2 * D, D), jnp.float32).at[:, :32].set(e_w1.T)
    eb1 = jnp.zeros((1, D), jnp.float32).at[0, :32].set(e_b1)
    ew2 = jnp.zeros((D, D), jnp.float32).at[:32, :32].set(e_w2.T)
    eb2 = jnp.zeros((1, D), jnp.float32).at[0, :32].set(e_b2)
    ew3 = jnp.zeros((D, D), jnp.float32).at[:32, :1].set(e_w3.T)
    eb3 = jnp.zeros((1, D), jnp.float32).at[0, :1].set(e_b3)

    pooled = [None] * P
    pooled[0] = _bag_sum_pieces[0](x_w, x_b, tab)
    prev = None
    for p in range(P):
        if p + 1 < P:
            pooled[p + 1] = _bag_sum_pieces[p + 1](x_w, x_b, tab)
        prev = _heads_piece(p, pooled[p], w1t, b1, ew1, eb1, ew2, eb2,
                            ew3, eb3, prev)
    y, zfull = prev
    return (y, zfull[:, :1])


# speedup vs baseline: 1.0829x; 1.0829x over previous
import functools

import jax
import jax.numpy as jnp
from jax import lax
from jax.experimental import pallas as pl
from jax.experimental.pallas import tpu as pltpu
from jax.experimental.pallas import tpu_sc as plsc

B = 4096
L = 200
V = 40961
D = 128
C0 = 104
C1 = 96
NW = 32
P = 1
NP = B // P
ROWS = 512


def _make_bag_sum(piece):
    spw = NP // NW

    def body(xw_hbm, xb_hbm, emb_hbm, out_hbm,
             idxw_v, idxb_v, rb0, rb1, out_v, sem0, sem1):
        w = lax.axis_index("s") * 2 + lax.axis_index("c")
        base = piece * NP + w * spw

        pltpu.sync_copy(xw_hbm.at[pl.ds(base, spw)], idxw_v)
        pltpu.sync_copy(xb_hbm.at[pl.ds(base, spw)], idxb_v)

        rbufs = (rb0, rb1)
        sems = (sem0, sem1)
        counts = (C0, C1)
        offs = (0, C0)

        def wait(b):
            pltpu.make_async_copy(
                emb_hbm.at[idxw_v.at[0, pl.ds(offs[b], counts[b])]],
                rbufs[b], sems[b]).wait()

        def accum(b, init):
            rbuf = rbufs[b]

            def rbody(j, a):
                word = rbuf[j, :]
                lo = jax.lax.bitcast_convert_type(word << 16, jnp.float32)
                hi = jax.lax.bitcast_convert_type(
                    word & jnp.int32(-65536), jnp.float32)
                return (a[0] + lo, a[1] + hi)

            return lax.fori_loop(0, counts[b], rbody, init, unroll=8)

        zeros = (jnp.zeros((D // 2,), jnp.float32),
                 jnp.zeros((D // 2,), jnp.float32))
        nch = 2 * spw

        def phase(idx_v, bag):
            def issue(c, b):
                pltpu.make_async_copy(
                    emb_hbm.at[idx_v.at[c >> 1, pl.ds(offs[b], counts[b])]],
                    rbufs[b], sems[b]).start()

            issue(0, 0)
            issue(1, 1)

            def outer(i, _):
                c = 2 * i
                wait(0)
                acc = accum(0, zeros)

                @pl.when(c + 2 < nch)
                def _():
                    issue(c + 2, 0)

                wait(1)
                acc = accum(1, acc)

                @pl.when(c + 3 < nch)
                def _():
                    issue(c + 3, 1)

                out_v[i, pl.ds(bag * D, D // 2)] = acc[0]
                out_v[i, pl.ds(bag * D + D // 2, D // 2)] = acc[1]
                return 0

            lax.fori_loop(0, spw, outer, 0)

        phase(idxw_v, 0)
        phase(idxb_v, 1)

        pltpu.sync_copy(out_v, out_hbm.at[pl.ds(w * spw, spw)])

    return functools.partial(
        pl.kernel,
        out_type=jax.ShapeDtypeStruct((NP, 2 * D), jnp.float32),
        mesh=plsc.VectorSubcoreMesh(core_axis_name="c", subcore_axis_name="s"),
        scratch_types=[
            pltpu.VMEM((spw, L), jnp.int32),
            pltpu.VMEM((spw, L), jnp.int32),
            pltpu.VMEM((C0, D // 2), jnp.int32),
            pltpu.VMEM((C1, D // 2), jnp.int32),
            pltpu.VMEM((spw, 2 * D), jnp.float32),
            pltpu.SemaphoreType.DMA,
            pltpu.SemaphoreType.DMA,
        ],
        compiler_params=pltpu.CompilerParams(use_tc_tiling_on_sc=False),
    )(body)


_bag_sum_pieces = [_make_bag_sum(p) for p in range(P)]


def _heads_piece(p, pooled, w1t, b1, ew1, eb1, ew2, eb2, ew3, eb3, prev):
    grid = (NP // ROWS,)
    off = p * (NP // ROWS)
    full = lambda shape: pl.BlockSpec(shape, lambda i: (0, 0))
    in_specs = [
        pl.BlockSpec((ROWS, 2 * D), lambda i: (i, 0)),
        full((2 * D, 4096)),
        full((1, 4096)),
        full((2 * D, D)),
        full((1, D)),
        full((D, D)),
        full((1, D)),
        full((D, D)),
        full((1, D)),
    ]
    args = [pooled, w1t, b1, ew1, eb1, ew2, eb2, ew3, eb3]
    aliases = {}
    if prev is not None:
        in_specs += [pl.BlockSpec(memory_space=pl.ANY),
                     pl.BlockSpec(memory_space=pl.ANY)]
        args += [prev[0], prev[1]]
        aliases = {9: 0, 10: 1}

    def body(*refs):
        (pooled_ref, w1t_ref, b1_ref, ew1_ref, eb1_ref,
         ew2_ref, eb2_ref, ew3_ref, eb3_ref) = refs[:9]
        y_ref, z_ref = refs[len(in_specs):]
        h = jnp.maximum(pooled_ref[...], 0.0)
        hp = jax.lax.Precision.DEFAULT
        y_ref[...] = (jnp.dot(h, w1t_ref[...], precision=hp,
                              preferred_element_type=jnp.float32)
                      + b1_ref[...])
        z1 = jnp.maximum(jnp.dot(h, ew1_ref[...], precision=hp,
                                 preferred_element_type=jnp.float32)
                         + eb1_ref[...], 0.0)
        z2 = jnp.maximum(jnp.dot(z1, ew2_ref[...], precision=hp,
                                 preferred_element_type=jnp.float32)
                         + eb2_ref[...], 0.0)
        z_ref[...] = (jnp.dot(z2, ew3_ref[...], precision=hp,
                              preferred_element_type=jnp.float32)
                      + eb3_ref[...])

    return pl.pallas_call(
        body,
        grid=grid,
        in_specs=in_specs,
        out_specs=[
            pl.BlockSpec((ROWS, 4096), lambda i: (off + i, 0)),
            pl.BlockSpec((ROWS, D), lambda i: (off + i, 0)),
        ],
        out_shape=[
            jax.ShapeDtypeStruct((B, 4096), jnp.float32),
            jax.ShapeDtypeStruct((B, D), jnp.float32),
        ],
        input_output_aliases=aliases,
    )(*args)


def kernel(x_w, x_b, emb, m_w1, m_b1, e_w1, e_b1, e_w2, e_b2, e_w3, e_b3):
    r = jax.lax.bitcast_convert_type(emb, jnp.uint32) + jnp.uint32(0x8000)
    w = (r[:, :64] >> 16) | (r[:, 64:] & jnp.uint32(0xFFFF0000))
    tab = jax.lax.bitcast_convert_type(w, jnp.int32)

    w1t = m_w1.T
    b1 = m_b1.reshape(1, 4096)
    ew1 = jnp.zeros((2 * D, D), jnp.float32).at[:, :32].set(e_w1.T)
    eb1 = jnp.zeros((1, D), jnp.float32).at[0, :32].set(e_b1)
    ew2 = jnp.zeros((D, D), jnp.float32).at[:32, :32].set(e_w2.T)
    eb2 = jnp.zeros((1, D), jnp.float32).at[0, :32].set(e_b2)
    ew3 = jnp.zeros((D, D), jnp.float32).at[:32, :1].set(e_w3.T)
    eb3 = jnp.zeros((1, D), jnp.float32).at[0, :1].set(e_b3)

    pooled = [None] * P
    pooled[0] = _bag_sum_pieces[0](x_w, x_b, tab)
    prev = None
    for p in range(P):
        if p + 1 < P:
            pooled[p + 1] = _bag_sum_pieces[p + 1](x_w, x_b, tab)
        prev = _heads_piece(p, pooled[p], w1t, b1, ew1, eb1, ew2, eb2,
                            ew3, eb3, prev)
    y, zfull = prev
    return (y, zfull[:, :1])
